# Initial kernel scaffold; baseline (speedup 1.0000x reference)
import jax
import jax.numpy as jnp
from jax.experimental import pallas as pl


def kernel(inputs, features, W_nb, b_nb, W_self, b_self, W_a1, b_a1, W_a2, b_a2):
    raise NotImplementedError("write your pallas kernel here")


# speedup vs baseline: 7.7112x; 7.7112x over previous
import functools

import jax
import jax.numpy as jnp
from jax import lax
from jax.experimental import pallas as pl
from jax.experimental.pallas import tpu as pltpu
from jax.experimental.pallas import tpu_sc as plsc

N = 10000
D = 128
H1 = 32
H2 = 16

NC = 2
NS = 16
NW = NC * NS
L = 16

CHUNK = 1024
IPB = 128
NIDX = CHUNK // IPB


def _tables_body(f_ref, wnb_ref, bnb_ref, wself_ref, bself_ref,
                 wa1t_ref, wa1b_ref, ba1_ref, g1_ref, g2_ref):
    x = f_ref[...]
    h1 = jnp.maximum(
        jnp.dot(x, wnb_ref[...], preferred_element_type=jnp.float32)
        + bnb_ref[...], 0.0)
    h2 = jnp.maximum(
        jnp.dot(x, wself_ref[...], preferred_element_type=jnp.float32)
        + bself_ref[...], 0.0)
    g1_ref[...] = (jnp.dot(h1, wa1t_ref[...], preferred_element_type=jnp.float32)
                   + ba1_ref[...])
    g2_ref[...] = jnp.dot(h2, wa1b_ref[...], preferred_element_type=jnp.float32)


def _node_tables(features, W_nb, b_nb, W_self, b_self, W_a1, b_a1):
    n = features.shape[0]
    blk = 1000
    grid = n // blk
    full = lambda shape: pl.BlockSpec(shape, lambda i: (0, 0))
    return pl.pallas_call(
        _tables_body,
        grid=(grid,),
        in_specs=[
            pl.BlockSpec((blk, D), lambda i: (i, 0)),
            full((D, H1)), full((1, H1)),
            full((D, H1)), full((1, H1)),
            full((H1, H2)), full((H1, H2)), full((1, H2)),
        ],
        out_specs=[
            pl.BlockSpec((blk, H2), lambda i: (i, 0)),
            pl.BlockSpec((blk, H2), lambda i: (i, 0)),
        ],
        out_shape=[
            jax.ShapeDtypeStruct((n, H2), jnp.float32),
            jax.ShapeDtypeStruct((n, H2), jnp.float32),
        ],
    )(features, W_nb, b_nb.reshape(1, H1), W_self, b_self.reshape(1, H1),
      W_a1[:H1], W_a1[H1:], b_a1.reshape(1, H2))


def _edge_body(epw, src_hbm, dst_hbm, g1_hbm, g2_hbm, pack_hbm, out_hbm,
               src_v, dst_v, r1_v, r2_v, out_v, pk_v, sem1, sem2):
    wid = lax.axis_index("s") * NC + lax.axis_index("c")
    base = wid * epw
    nchunk = epw // CHUNK

    pltpu.sync_copy(pack_hbm, pk_v)
    iota = lax.iota(jnp.int32, L)
    w2s = [pk_v[k, :] for k in range(L)]
    b2s = pk_v[L, :]

    def chunk_body(ci, carry):
        row0 = wid * (epw // IPB) + ci * NIDX
        pltpu.sync_copy(src_hbm.at[pl.ds(row0, NIDX)], src_v)
        pltpu.sync_copy(dst_hbm.at[pl.ds(row0, NIDX)], dst_v)
        d1 = [pltpu.async_copy(g1_hbm.at[src_v.at[j]],
                               r1_v.at[pl.ds(j * IPB, IPB), :], sem1)
              for j in range(NIDX)]
        d2 = [pltpu.async_copy(g2_hbm.at[dst_v.at[j]],
                               r2_v.at[pl.ds(j * IPB, IPB), :], sem2)
              for j in range(NIDX)]
        for d in d1:
            d.wait()
        for d in d2:
            d.wait()

        def group_body(g, carry2):
            ridx = g * L + iota
            acc = b2s
            for k in range(L):
                a = plsc.load_gather(r1_v, [ridx, jnp.full((L,), k, jnp.int32)])
                b = plsc.load_gather(r2_v, [ridx, jnp.full((L,), k, jnp.int32)])
                acc = acc + jnp.maximum(a + b, 0.0) * w2s[k]
            out_v[pl.ds(g * L, L)] = acc
            return carry2

        lax.fori_loop(0, CHUNK // L, group_body, 0)
        pltpu.sync_copy(out_v, out_hbm.at[pl.ds(base + ci * CHUNK, CHUNK)])
        return carry

    lax.fori_loop(0, nchunk, chunk_body, 0)


def _edge_call(e_pad, src2d, dst2d, g1, g2, pack):
    epw = e_pad // NW
    mesh = plsc.VectorSubcoreMesh(core_axis_name="c", subcore_axis_name="s")
    return pl.kernel(
        functools.partial(_edge_body, epw),
        out_type=jax.ShapeDtypeStruct((e_pad,), jnp.float32),
        mesh=mesh,
        compiler_params=pltpu.CompilerParams(
            needs_layout_passes=False, use_tc_tiling_on_sc=False),
        scratch_types=[
            pltpu.VMEM((NIDX, IPB), jnp.int32),
            pltpu.VMEM((NIDX, IPB), jnp.int32),
            pltpu.VMEM((CHUNK, L), jnp.float32),
            pltpu.VMEM((CHUNK, L), jnp.float32),
            pltpu.VMEM((CHUNK,), jnp.float32),
            pltpu.VMEM((L + 2, L), jnp.float32),
            pltpu.SemaphoreType.DMA,
            pltpu.SemaphoreType.DMA,
        ],
    )(src2d, dst2d, g1, g2, pack)


def kernel(inputs, features, W_nb, b_nb, W_self, b_self, W_a1, b_a1, W_a2, b_a2):
    e = inputs.shape[0]
    grain = NW * CHUNK
    e_pad = ((e + grain - 1) // grain) * grain

    src = inputs[:, 0].astype(jnp.int32)
    dst = inputs[:, 1].astype(jnp.int32)
    pad = e_pad - e
    src = jnp.concatenate([src, jnp.zeros((pad,), jnp.int32)])
    dst = jnp.concatenate([dst, jnp.zeros((pad,), jnp.int32)])
    src2d = src.reshape(e_pad // IPB, IPB)
    dst2d = dst.reshape(e_pad // IPB, IPB)

    g1, g2 = _node_tables(features, W_nb, b_nb, W_self, b_self, W_a1, b_a1)
    pack = jnp.concatenate([
        jnp.broadcast_to(W_a2[:, 0:1], (L, L)),
        jnp.broadcast_to(b_a2, (2, L)),
    ])

    out = _edge_call(e_pad, src2d, dst2d, g1, g2, pack)
    return out[:e, None]


# speedup vs baseline: 10.4081x; 1.3497x over previous
import functools

import jax
import jax.numpy as jnp
from jax import lax
from jax.experimental import pallas as pl
from jax.experimental.pallas import tpu as pltpu
from jax.experimental.pallas import tpu_sc as plsc

N = 10000
D = 128
H1 = 32
H2 = 16

NC = 2
NS = 16
NW = NC * NS
L = 16

CHUNK = 512
NBUF = 2
IPB = 128
NIDX = CHUNK // IPB


def _tables_body(f_ref, wnb_ref, bnb_ref, wself_ref, bself_ref,
                 wa1t_ref, wa1b_ref, ba1_ref, g1_ref, g2_ref):
    x = f_ref[...]
    h1 = jnp.maximum(
        jnp.dot(x, wnb_ref[...], preferred_element_type=jnp.float32)
        + bnb_ref[...], 0.0)
    h2 = jnp.maximum(
        jnp.dot(x, wself_ref[...], preferred_element_type=jnp.float32)
        + bself_ref[...], 0.0)
    g1_ref[...] = (jnp.dot(h1, wa1t_ref[...], preferred_element_type=jnp.float32)
                   + ba1_ref[...])
    g2_ref[...] = jnp.dot(h2, wa1b_ref[...], preferred_element_type=jnp.float32)


def _node_tables(features, W_nb, b_nb, W_self, b_self, W_a1, b_a1):
    n = features.shape[0]
    blk = 1000
    grid = n // blk
    full = lambda shape: pl.BlockSpec(shape, lambda i: (0, 0))
    return pl.pallas_call(
        _tables_body,
        grid=(grid,),
        in_specs=[
            pl.BlockSpec((blk, D), lambda i: (i, 0)),
            full((D, H1)), full((1, H1)),
            full((D, H1)), full((1, H1)),
            full((H1, H2)), full((H1, H2)), full((1, H2)),
        ],
        out_specs=[
            pl.BlockSpec((blk, H2), lambda i: (i, 0)),
            pl.BlockSpec((blk, H2), lambda i: (i, 0)),
        ],
        out_shape=[
            jax.ShapeDtypeStruct((n, H2), jnp.float32),
            jax.ShapeDtypeStruct((n, H2), jnp.float32),
        ],
    )(features, W_nb, b_nb.reshape(1, H1), W_self, b_self.reshape(1, H1),
      W_a1[:H1], W_a1[H1:], b_a1.reshape(1, H2))


def _edge_body(epw, src_hbm, dst_hbm, g1_hbm, g2_hbm, pack_hbm, out_hbm,
               src_v, dst_v, r1_v, r2_v, out_v, pk_v, sem1, sem2):
    wid = lax.axis_index("s") * NC + lax.axis_index("c")
    base = wid * epw
    nchunk = epw // CHUNK

    pltpu.sync_copy(pack_hbm, pk_v)
    iota = lax.iota(jnp.int32, L)
    w2s = [pk_v[k, :] for k in range(L)]
    b2s = pk_v[L, :]

    def issue(ci, buf):
        row0 = wid * (epw // IPB) + ci * NIDX
        pltpu.sync_copy(src_hbm.at[pl.ds(row0, NIDX)], src_v.at[buf])
        pltpu.sync_copy(dst_hbm.at[pl.ds(row0, NIDX)], dst_v.at[buf])
        for j in range(NIDX):
            pltpu.async_copy(g1_hbm.at[src_v.at[buf, j]],
                             r1_v.at[buf, pl.ds(j * IPB, IPB), :], sem1.at[buf])
            pltpu.async_copy(g2_hbm.at[dst_v.at[buf, j]],
                             r2_v.at[buf, pl.ds(j * IPB, IPB), :], sem2.at[buf])

    def drain(buf):
        for j in range(NIDX):
            pltpu.make_async_copy(g1_hbm.at[src_v.at[buf, j]],
                                  r1_v.at[buf, pl.ds(j * IPB, IPB), :],
                                  sem1.at[buf]).wait()
            pltpu.make_async_copy(g2_hbm.at[dst_v.at[buf, j]],
                                  r2_v.at[buf, pl.ds(j * IPB, IPB), :],
                                  sem2.at[buf]).wait()

    for b in range(NBUF):
        issue(b, b)

    def chunk_body(ci, carry):
        buf = lax.rem(ci, NBUF)
        drain(buf)

        def group_body(g, carry2):
            ridx = g * L + iota
            acc = b2s
            for k in range(L):
                a = plsc.load_gather(
                    r1_v.at[buf], [ridx, jnp.full((L,), k, jnp.int32)])
                b = plsc.load_gather(
                    r2_v.at[buf], [ridx, jnp.full((L,), k, jnp.int32)])
                acc = acc + jnp.maximum(a + b, 0.0) * w2s[k]
            out_v[pl.ds(g * L, L)] = acc
            return carry2

        lax.fori_loop(0, CHUNK // L, group_body, 0)
        pltpu.sync_copy(out_v, out_hbm.at[pl.ds(base + ci * CHUNK, CHUNK)])

        @pl.when(ci + NBUF < nchunk)
        def _():
            issue(ci + NBUF, buf)

        return carry

    lax.fori_loop(0, nchunk, chunk_body, 0)


def _edge_call(e_pad, src2d, dst2d, g1, g2, pack):
    epw = e_pad // NW
    mesh = plsc.VectorSubcoreMesh(core_axis_name="c", subcore_axis_name="s")
    return pl.kernel(
        functools.partial(_edge_body, epw),
        out_type=jax.ShapeDtypeStruct((e_pad,), jnp.float32),
        mesh=mesh,
        compiler_params=pltpu.CompilerParams(
            needs_layout_passes=False, use_tc_tiling_on_sc=False),
        scratch_types=[
            pltpu.VMEM((NBUF, NIDX, IPB), jnp.int32),
            pltpu.VMEM((NBUF, NIDX, IPB), jnp.int32),
            pltpu.VMEM((NBUF, CHUNK, L), jnp.float32),
            pltpu.VMEM((NBUF, CHUNK, L), jnp.float32),
            pltpu.VMEM((CHUNK,), jnp.float32),
            pltpu.VMEM((L + 2, L), jnp.float32),
            pltpu.SemaphoreType.DMA((NBUF,)),
            pltpu.SemaphoreType.DMA((NBUF,)),
        ],
    )(src2d, dst2d, g1, g2, pack)


def kernel(inputs, features, W_nb, b_nb, W_self, b_self, W_a1, b_a1, W_a2, b_a2):
    e = inputs.shape[0]
    grain = NW * CHUNK
    e_pad = ((e + grain - 1) // grain) * grain

    src = inputs[:, 0].astype(jnp.int32)
    dst = inputs[:, 1].astype(jnp.int32)
    pad = e_pad - e
    src = jnp.concatenate([src, jnp.zeros((pad,), jnp.int32)])
    dst = jnp.concatenate([dst, jnp.zeros((pad,), jnp.int32)])
    src2d = src.reshape(e_pad // IPB, IPB)
    dst2d = dst.reshape(e_pad // IPB, IPB)

    g1, g2 = _node_tables(features, W_nb, b_nb, W_self, b_self, W_a1, b_a1)
    pack = jnp.concatenate([
        jnp.broadcast_to(W_a2[:, 0:1], (L, L)),
        jnp.broadcast_to(b_a2, (2, L)),
    ])

    out = _edge_call(e_pad, src2d, dst2d, g1, g2, pack)
    return out[:e, None]
